# baseline (speedup 1.0000x reference)
import functools

import jax
import jax.numpy as jnp
from jax import lax
from jax.experimental import pallas as pl
from jax.experimental.pallas import tpu as pltpu

_N_EMB = 1024
_DIM = 64
_TOKENS = 16 * 1024
_BLK = 2048
_GRID = _TOKENS // _BLK


def _vq_body(x_ref, e_ref, q_ref, idx_ref, losssum_ref):
    i = pl.program_id(0)
    x = x_ref[...]
    e = e_ref[...]
    x2 = jnp.sum(x * x, axis=1, keepdims=True)
    e2 = jnp.sum(e * e, axis=1)
    xe = lax.dot_general(x, e, (((1,), (1,)), ((), ())),
                         preferred_element_type=jnp.float32)
    d = x2 + e2[None, :] - 2.0 * xe
    m = jnp.min(d, axis=1, keepdims=True)
    col = lax.broadcasted_iota(jnp.int32, d.shape, 1)
    idx = jnp.min(jnp.where(d == m, col, _N_EMB), axis=1)
    idx_ref[...] = idx.reshape(1, 1, _BLK)
    onehot = (col == idx[:, None]).astype(jnp.float32)
    q_ref[...] = lax.dot_general(onehot, e, (((1,), (0,)), ((), ())),
                                 preferred_element_type=jnp.float32)

    @pl.when(i == 0)
    def _init():
        losssum_ref[0, 0] = 0.0

    losssum_ref[0, 0] += jnp.sum(m)


@jax.jit
def kernel(inputs, embeddings):
    flat = inputs.reshape(_TOKENS, _DIM)
    q, idx3, losssum = pl.pallas_call(
        _vq_body,
        grid=(_GRID,),
        in_specs=[
            pl.BlockSpec((_BLK, _DIM), lambda i: (i, 0)),
            pl.BlockSpec((_N_EMB, _DIM), lambda i: (0, 0)),
        ],
        out_specs=[
            pl.BlockSpec((_BLK, _DIM), lambda i: (i, 0)),
            pl.BlockSpec((1, 1, _BLK), lambda i: (i, 0, 0)),
            pl.BlockSpec(memory_space=pltpu.SMEM),
        ],
        out_shape=[
            jax.ShapeDtypeStruct((_TOKENS, _DIM), jnp.float32),
            jax.ShapeDtypeStruct((_GRID, 1, _BLK), jnp.int32),
            jax.ShapeDtypeStruct((1, 1), jnp.float32),
        ],
    )(flat, embeddings)
    loss = losssum[0, 0] / jnp.float32(_TOKENS * _DIM)
    quantized_st = q.reshape(inputs.shape)
    return quantized_st, loss, idx3.reshape(_TOKENS)[:, None]
